# baseline (speedup 1.0000x reference)
import functools

import jax
import jax.numpy as jnp
from jax import lax
from jax.experimental import pallas as pl
from jax.experimental.pallas import tpu as pltpu
from jax.experimental.pallas import tpu_sc as plsc

B = 16384
D = 64
NC = 2
NS = 16
NW = NC * NS
BPW = B // NW
CH = 128
NCH = BPW // CH


def _embed_body(table_hbm, idx_hbm, out_hbm, idx_v, rows_v, sem):
    wid = lax.axis_index("s") * NC + lax.axis_index("c")
    base = wid * BPW
    pltpu.sync_copy(idx_hbm.at[pl.ds(wid * NCH, NCH)], idx_v)
    copies = []
    for j in range(NCH):
        copies.append(
            pltpu.async_copy(
                table_hbm.at[idx_v.at[j]],
                rows_v.at[pl.ds(j * CH, CH)],
                sem,
            )
        )
    for c in copies:
        c.wait()
    pltpu.sync_copy(rows_v, out_hbm.at[pl.ds(base, BPW)])


@jax.jit
def _embed(table, idx2d):
    mesh = plsc.VectorSubcoreMesh(core_axis_name="c", subcore_axis_name="s")
    run = functools.partial(
        pl.kernel,
        mesh=mesh,
        out_type=jax.ShapeDtypeStruct((B, D), jnp.float32),
        scratch_types=[
            pltpu.VMEM((NCH, CH), jnp.int32),
            pltpu.VMEM((BPW, D), jnp.float32),
            pltpu.SemaphoreType.DMA,
        ],
        compiler_params=pltpu.CompilerParams(use_tc_tiling_on_sc=False),
    )(_embed_body)
    return run(table, idx2d)


def kernel(class_ids, table):
    idx2d = class_ids.astype(jnp.int32).reshape(NW * NCH, CH)
    out = _embed(table, idx2d)
    return out.reshape(B, 1, D)


# speedup vs baseline: 1.7298x; 1.7298x over previous
import functools

import jax
import jax.numpy as jnp
from jax import lax
from jax.experimental import pallas as pl
from jax.experimental.pallas import tpu as pltpu
from jax.experimental.pallas import tpu_sc as plsc

B = 16384
D = 64
NC = 2
NS = 16
NW = NC * NS
BPW = B // NW
L = 16
NCHUNK = BPW // L


def _embed_body(table_hbm, idx_hbm, out_hbm, idx_v, rows_v, sem):
    wid = lax.axis_index("s") * NC + lax.axis_index("c")
    base = wid * BPW
    pltpu.sync_copy(idx_hbm.at[pl.ds(base, BPW)], idx_v)
    lane = lax.iota(jnp.int32, L)

    @pl.loop(0, NCHUNK)
    def _chunks(c):
        v = idx_v[pl.ds(c * L, L)]
        for j in range(L):
            s = jnp.sum(jnp.where(lane == j, v, 0))
            pltpu.async_copy(
                table_hbm.at[pl.ds(s, 1)],
                rows_v.at[pl.ds(c * L + j, 1)],
                sem,
            )

    pltpu.make_async_copy(table_hbm.at[pl.ds(0, BPW)], rows_v, sem).wait()
    pltpu.sync_copy(rows_v, out_hbm.at[pl.ds(base, BPW)])


@jax.jit
def _embed(table, idx):
    mesh = plsc.VectorSubcoreMesh(core_axis_name="c", subcore_axis_name="s")
    run = functools.partial(
        pl.kernel,
        mesh=mesh,
        out_type=jax.ShapeDtypeStruct((B, D), jnp.float32),
        scratch_types=[
            pltpu.VMEM((BPW,), jnp.int32),
            pltpu.VMEM((BPW, D), jnp.float32),
            pltpu.SemaphoreType.DMA,
        ],
        compiler_params=pltpu.CompilerParams(needs_layout_passes=False),
    )(_embed_body)
    return run(table, idx)


def kernel(class_ids, table):
    out = _embed(table, class_ids.astype(jnp.int32))
    return out.reshape(B, 1, D)


# speedup vs baseline: 2.9322x; 1.6951x over previous
import functools

import jax
import jax.numpy as jnp
from jax import lax
from jax.experimental import pallas as pl
from jax.experimental.pallas import tpu as pltpu
from jax.experimental.pallas import tpu_sc as plsc

B = 16384
D = 64
NC = 2
NS = 16
NW = NC * NS
L = 16
SLABW = 256
RAG_S = 3906
RAGW = 65
NG_IDX = B // L
RING = 64


def _embed_body(tab_t, idx_hbm, out, idx_v, sel_i, sel_b, slab2, rag_v, rows_v,
                semi, semd, semo):
    wid = lax.axis_index("s") * NC + lax.axis_index("c")
    lane = lax.iota(jnp.int32, L)
    pltpu.async_copy(idx_hbm, idx_v, semi).wait()

    @pl.loop(0, NG_IDX, init_carry=jnp.zeros((L,), jnp.int32))
    def cnt_v(g, base):
        v = idx_v[pl.ds(g * L, L)]
        m = ((v >> 8) & 31) == wid
        cs = plsc.cumsum(jnp.where(m, 1, 0))
        pos = base + cs - 1
        plsc.store_scatter(sel_i, [pos], v, mask=m)
        plsc.store_scatter(sel_b, [pos], lane + g * L, mask=m)
        return base + plsc.all_reduce_population_count(m)

    cnt = jnp.max(cnt_v)
    ng = (cnt + L - 1) // L
    nk = 122 + jnp.where(wid < 2, 1, 0)

    def issue(k):
        s = wid + NW * k
        col0 = pl.multiple_of(s * SLABW, 128)
        pltpu.async_copy(
            tab_t.at[:, pl.ds(col0, SLABW)], slab2.at[k & 1], semd.at[k & 1]
        )

    def wait(k):
        s = wid + NW * k
        col0 = pl.multiple_of(s * SLABW, 128)
        pltpu.make_async_copy(
            tab_t.at[:, pl.ds(col0, SLABW)], slab2.at[k & 1], semd.at[k & 1]
        ).wait()

    def process(buf, s, e0):
        @pl.loop(0, ng, init_carry=e0)
        def e_out(g, e):
            vsel = sel_i[pl.ds(g * L, L)]
            vb = sel_b[pl.ds(g * L, L)]
            m0 = ((vsel >> 8) == s) & ((lane + g * L) < cnt_v)

            def cond(c):
                m, _ = c
                return jnp.any(m)

            def body(c):
                m, e = c
                j = plsc.all_reduce_ffs(m)
                scol = jnp.sum(jnp.where(lane == j, vsel, 0)) - s * SLABW
                colsp = jnp.broadcast_to(scol, (L,))
                sb = jnp.sum(jnp.where(lane == j, vb, 0))
                slot = e & (RING - 1)

                @pl.when(e >= RING)
                def _():
                    pltpu.make_async_copy(
                        out.at[pl.ds(0, 1), :], rows_v.at[pl.ds(0, 1), :], semo
                    ).wait()

                for q in range(D // L):
                    val = plsc.load_gather(buf, [lane + q * L, colsp])
                    rows_v[slot, pl.ds(q * L, L)] = val
                pltpu.async_copy(
                    rows_v.at[pl.ds(slot, 1), :], out.at[pl.ds(sb, 1), :], semo
                )
                return m & (lane != j), e + 1

            _, e1 = lax.while_loop(cond, body, (m0, e))
            return e1

        return e_out

    issue(0)

    @pl.loop(0, nk, init_carry=jnp.int32(0))
    def e_fin(k, e):
        @pl.when(k + 1 < nk)
        def _():
            issue(k + 1)

        wait(k)
        return process(slab2.at[k & 1], wid + NW * k, e)

    def do_ragged(e):
        pltpu.sync_copy(tab_t.at[:, pl.ds(RAG_S * SLABW, RAGW)], rag_v)
        return process(rag_v, RAG_S, e)

    e_fin2 = lax.cond(wid == (RAG_S & 31), do_ragged, lambda e: e, e_fin)

    @pl.loop(0, jnp.minimum(e_fin2, RING))
    def _(i):
        pltpu.make_async_copy(
            out.at[pl.ds(0, 1), :], rows_v.at[pl.ds(0, 1), :], semo
        ).wait()


@jax.jit
def _embed(table_t, idx):
    mesh = plsc.VectorSubcoreMesh(core_axis_name="c", subcore_axis_name="s")
    run = functools.partial(
        pl.kernel,
        mesh=mesh,
        out_type=jax.ShapeDtypeStruct((B, D), jnp.float32),
        scratch_types=[
            pltpu.VMEM((B,), jnp.int32),
            pltpu.VMEM((B,), jnp.int32),
            pltpu.VMEM((B,), jnp.int32),
            pltpu.VMEM((2, D, SLABW), jnp.float32),
            pltpu.VMEM((D, RAGW), jnp.float32),
            pltpu.VMEM((RING, D), jnp.float32),
            pltpu.SemaphoreType.DMA,
            pltpu.SemaphoreType.DMA((2,)),
            pltpu.SemaphoreType.DMA,
        ],
        compiler_params=pltpu.CompilerParams(needs_layout_passes=False),
    )(_embed_body)
    return run(table_t, idx)


def kernel(class_ids, table):
    out = _embed(table.T, class_ids.astype(jnp.int32))
    return out.reshape(B, 1, D)


# speedup vs baseline: 3.9996x; 1.3640x over previous
import functools

import jax
import jax.numpy as jnp
from jax import lax
from jax.experimental import pallas as pl
from jax.experimental.pallas import tpu as pltpu
from jax.experimental.pallas import tpu_sc as plsc

B = 16384
D = 64
NC = 2
NS = 16
NW = NC * NS
L = 16
SLABW = 512
RAG_S = 1953
RAG_K = RAG_S >> 5
RAGW = 65
NG_IDX = B // L
RING = 64
SENT = 0x7FFFFFFF


def _embed_body(tab_t, idx_hbm, out, idx_v, sel_v, slab2, rag_v, rows_v,
                semi, semd, semo):
    wid = lax.axis_index("s") * NC + lax.axis_index("c")
    lane = lax.iota(jnp.int32, L)
    pltpu.async_copy(idx_hbm, idx_v, semi).wait()

    @pl.loop(0, NG_IDX, init_carry=jnp.zeros((L,), jnp.int32), unroll=4)
    def cnt_v(g, base):
        v = idx_v[pl.ds(g * L, L)]
        m = ((v >> 9) & 31) == wid
        packed = ((v >> 14) << 23) | ((v & 511) << 14) | (lane + g * L)
        cs = plsc.cumsum(jnp.where(m, 1, 0))
        plsc.store_scatter(sel_v, [base + cs - 1], packed, mask=m)
        return base + plsc.all_reduce_population_count(m)

    cnt = jnp.max(cnt_v)
    plsc.store_scatter(sel_v, [cnt + lane],
                       jnp.full((L,), SENT, jnp.int32), mask=lane == lane)
    ng = (cnt + L - 1) // L
    nk = 61 + jnp.where(wid < 1, 1, 0)

    def issue(k):
        s = wid + NW * k
        col0 = pl.multiple_of(s * SLABW, 128)
        pltpu.async_copy(
            tab_t.at[:, pl.ds(col0, SLABW)], slab2.at[k & 1], semd.at[k & 1]
        )

    def wait(k):
        s = wid + NW * k
        col0 = pl.multiple_of(s * SLABW, 128)
        pltpu.make_async_copy(
            tab_t.at[:, pl.ds(col0, SLABW)], slab2.at[k & 1], semd.at[k & 1]
        ).wait()

    def process(buf, kord, e0):
        @pl.loop(0, ng, init_carry=e0)
        def e_out(g, e):
            vsel = sel_v[pl.ds(g * L, L)]
            m0 = (vsel >> 23) == kord

            def cond(c):
                m, _ = c
                return jnp.any(m)

            def body(c):
                m, e = c
                j = plsc.all_reduce_ffs(m)
                sv = jnp.sum(jnp.where(lane == j, vsel, 0))
                scol = (sv >> 14) & 511
                sb = sv & 16383
                colsp = jnp.broadcast_to(scol, (L,))
                slot = e & (RING - 1)

                @pl.when(e >= RING)
                def _():
                    pltpu.make_async_copy(
                        out.at[pl.ds(0, 1), :], rows_v.at[pl.ds(0, 1), :], semo
                    ).wait()

                for q in range(D // L):
                    val = plsc.load_gather(buf, [lane + q * L, colsp])
                    rows_v[slot, pl.ds(q * L, L)] = val
                pltpu.async_copy(
                    rows_v.at[pl.ds(slot, 1), :], out.at[pl.ds(sb, 1), :], semo
                )
                return m & (lane != j), e + 1

            _, e1 = lax.while_loop(cond, body, (m0, e))
            return e1

        return e_out

    issue(0)

    @pl.loop(0, nk, init_carry=jnp.int32(0))
    def e_fin(k, e):
        @pl.when(k + 1 < nk)
        def _():
            issue(k + 1)

        wait(k)
        return process(slab2.at[k & 1], k, e)

    def do_ragged(e):
        pltpu.sync_copy(tab_t.at[:, pl.ds(RAG_S * SLABW, RAGW)], rag_v)
        return process(rag_v, RAG_K, e)

    e_fin2 = lax.cond(wid == (RAG_S & 31), do_ragged, lambda e: e, e_fin)

    @pl.loop(0, jnp.minimum(e_fin2, RING))
    def _(i):
        pltpu.make_async_copy(
            out.at[pl.ds(0, 1), :], rows_v.at[pl.ds(0, 1), :], semo
        ).wait()


@jax.jit
def _embed(table_t, idx):
    mesh = plsc.VectorSubcoreMesh(core_axis_name="c", subcore_axis_name="s")
    run = functools.partial(
        pl.kernel,
        mesh=mesh,
        out_type=jax.ShapeDtypeStruct((B, D), jnp.float32),
        scratch_types=[
            pltpu.VMEM((B,), jnp.int32),
            pltpu.VMEM((B + L,), jnp.int32),
            pltpu.VMEM((2, D, SLABW), jnp.float32),
            pltpu.VMEM((D, RAGW), jnp.float32),
            pltpu.VMEM((RING, D), jnp.float32),
            pltpu.SemaphoreType.DMA,
            pltpu.SemaphoreType.DMA((2,)),
            pltpu.SemaphoreType.DMA,
        ],
        compiler_params=pltpu.CompilerParams(needs_layout_passes=False),
    )(_embed_body)
    return run(table_t, idx)


def kernel(class_ids, table):
    out = _embed(table.T, class_ids.astype(jnp.int32))
    return out.reshape(B, 1, D)
